# Initial kernel scaffold; baseline (speedup 1.0000x reference)
import jax
import jax.numpy as jnp
from jax.experimental import pallas as pl


def kernel(node_type_ids, edge_index, edge_type_ids, anchor_index, numeric, boolean, categorical, node_emb, Wself, bself, Wrel, ce0, ce1, ce2, Wf, bf, Wb1, bb1, Wb2, bb2, Ws1, bs1, Ws2, bs2):
    raise NotImplementedError("write your pallas kernel here")


# speedup vs baseline: 10.9229x; 10.9229x over previous
import functools

import jax
import jax.numpy as jnp
from jax import lax
from jax.experimental import pallas as pl
from jax.experimental.pallas import tpu as pltpu
from jax.experimental.pallas import tpu_sc as plsc

N = 50000
E = 800000
H = 64
R = 4
L = 2

f32 = jnp.float32
i32 = jnp.int32

BN = 2000
NB = N // BN

NSC = 2
NTS = 16
CH = 128
KC = 56
PTC = 392
PT = PTC * CH
EPAD = NTS * PT
NGS = PTC // KC
A = 50176
ZR = A // NTS
DUMMY = N

_HI = lax.Precision.HIGHEST


def _dot(a, b):
    return jnp.dot(a, b, precision=_HI, preferred_element_type=f32)


def _k1_body(ids_ref, emb_ref, wp_ref, ns_ref, plo_ref, phi_ref):
    ids = ids_ref[0, 0, :]
    io = lax.broadcasted_iota(i32, (BN, H), 1)
    oh = jnp.where(ids[:, None] == io, 1.0, 0.0).astype(f32)
    ns = _dot(oh, emb_ref[...])
    ns_ref[...] = ns
    p = _dot(ns, wp_ref[...])
    for r in range(R):
        plo_ref[r] = p[:, r * H:r * H + 32]
        phi_ref[r] = p[:, r * H + 32:r * H + H]


def _k1(ids3, node_emb, wp0):
    return pl.pallas_call(
        _k1_body,
        grid=(NB,),
        in_specs=[
            pl.BlockSpec((1, 1, BN), lambda i: (i, 0, 0)),
            pl.BlockSpec((H, H), lambda i: (0, 0)),
            pl.BlockSpec((H, R * H), lambda i: (0, 0)),
        ],
        out_specs=[
            pl.BlockSpec((BN, H), lambda i: (i, 0)),
            pl.BlockSpec((R, BN, 32), lambda i: (0, i, 0)),
            pl.BlockSpec((R, BN, 32), lambda i: (0, i, 0)),
        ],
        out_shape=[
            jax.ShapeDtypeStruct((N, H), f32),
            jax.ShapeDtypeStruct((R, N, 32), f32),
            jax.ShapeDtypeStruct((R, N, 32), f32),
        ],
    )(ids3, node_emb, wp0)


def _sc_body(plo, phi, ridx, didx, zeros, out, acc, ridx_v, didx_v, rows_v,
             sem):
    c = lax.axis_index("c")
    s = lax.axis_index("s")
    pltpu.sync_copy(zeros, acc.at[pl.ds(s * ZR, ZR)])
    plsc.subcore_barrier()

    def run(tbl):
        def outer(g, carry):
            row0 = s * PTC + g * KC
            pltpu.sync_copy(ridx.at[pl.ds(row0, KC)], ridx_v)
            pltpu.sync_copy(didx.at[pl.ds(row0, KC)], didx_v)

            def inner(j, carry2):
                pltpu.async_copy(tbl.at[ridx_v.at[j]], rows_v, sem).wait()
                pltpu.sync_copy(rows_v, acc.at[didx_v.at[j]], add=True)
                return carry2

            return lax.fori_loop(0, KC, inner, carry)

        lax.fori_loop(0, NGS, outer, 0)

    @pl.when(c == 0)
    def _():
        run(plo)

    @pl.when(c == 1)
    def _():
        run(phi)

    plsc.subcore_barrier()

    @pl.when(c == 0)
    def _():
        pltpu.sync_copy(acc.at[pl.ds(s * ZR, ZR)], out.at[0, pl.ds(s * ZR, ZR)])

    @pl.when(c == 1)
    def _():
        pltpu.sync_copy(acc.at[pl.ds(s * ZR, ZR)], out.at[1, pl.ds(s * ZR, ZR)])


_sc_msgs = pl.kernel(
    _sc_body,
    out_type=jax.ShapeDtypeStruct((NSC, A, 32), f32),
    mesh=plsc.VectorSubcoreMesh(
        core_axis_name="c", subcore_axis_name="s", num_cores=NSC,
        num_subcores=NTS),
    scratch_types=[
        pltpu.VMEM_SHARED((A, 32), f32),
        pltpu.VMEM((KC, CH), i32),
        pltpu.VMEM((KC, CH), i32),
        pltpu.VMEM((CH, 32), f32),
        pltpu.SemaphoreType.DMA,
    ],
    compiler_params=pltpu.CompilerParams(use_tc_tiling_on_sc=False),
)


def _k2_body(ns_ref, m_ref, wsT_ref, bs_ref, wp_ref, nsn_ref, plo_ref,
             phi_ref):
    z = _dot(ns_ref[...], wsT_ref[...]) + bs_ref[...]
    msg = jnp.concatenate([m_ref[0], m_ref[1]], axis=1)
    nsn = jnp.maximum(z + msg, 0.0)
    nsn_ref[...] = nsn
    p = _dot(nsn, wp_ref[...])
    for r in range(R):
        plo_ref[r] = p[:, r * H:r * H + 32]
        phi_ref[r] = p[:, r * H + 32:r * H + H]


def _k2(ns, msgs, wsT, bs, wp):
    return pl.pallas_call(
        _k2_body,
        grid=(NB,),
        in_specs=[
            pl.BlockSpec((BN, H), lambda i: (i, 0)),
            pl.BlockSpec((NSC, BN, 32), lambda i: (0, i, 0)),
            pl.BlockSpec((H, H), lambda i: (0, 0)),
            pl.BlockSpec((1, H), lambda i: (0, 0)),
            pl.BlockSpec((H, R * H), lambda i: (0, 0)),
        ],
        out_specs=[
            pl.BlockSpec((BN, H), lambda i: (i, 0)),
            pl.BlockSpec((R, BN, 32), lambda i: (0, i, 0)),
            pl.BlockSpec((R, BN, 32), lambda i: (0, i, 0)),
        ],
        out_shape=[
            jax.ShapeDtypeStruct((N, H), f32),
            jax.ShapeDtypeStruct((R, N, 32), f32),
            jax.ShapeDtypeStruct((R, N, 32), f32),
        ],
    )(ns, msgs, wsT, bs, wp)


def _k3_body(ns_ref, m_ref, wsT_ref, bs_ref, aidx_ref, feat_ref, wfT_ref,
             bf_ref, wb1T_ref, bb1_ref, zb2_ref, bb2_ref, ws1T_ref, bs1_ref,
             ws2T_ref, bs2_ref, blog_ref, sev_ref, ssum, smax, sanc):
    i = pl.program_id(0)

    @pl.when(i == 0)
    def _():
        ssum[...] = jnp.zeros((8, H), f32)
        smax[...] = jnp.zeros((8, H), f32)
        sanc[...] = jnp.zeros((8, H), f32)
        blog_ref[...] = jnp.zeros((1, 8), f32)
        sev_ref[...] = jnp.zeros((1, 8), f32)

    z = _dot(ns_ref[...], wsT_ref[...]) + bs_ref[...]
    msg = jnp.concatenate([m_ref[0], m_ref[1]], axis=1)
    nsn = jnp.maximum(z + msg, 0.0)
    ssum[0:1, :] = ssum[0:1, :] + jnp.sum(nsn, axis=0, keepdims=True)
    smax[0:1, :] = jnp.maximum(smax[0:1, :], jnp.max(nsn, axis=0,
                                                     keepdims=True))
    aidx = aidx_ref[0, 0]
    rows = i * BN + lax.broadcasted_iota(i32, (BN, 1), 0)
    am = jnp.where(rows == aidx, 1.0, 0.0).astype(f32)
    sanc[0:1, :] = sanc[0:1, :] + jnp.sum(nsn * am, axis=0, keepdims=True)

    @pl.when(i == NB - 1)
    def _():
        gmean = ssum[0:1, :] * (1.0 / N)
        gmax = smax[0:1, :]
        anc = sanc[0:1, :]
        enc = jnp.maximum(_dot(feat_ref[...], wfT_ref[...]) + bf_ref[...],
                          0.0)
        comb = jnp.concatenate([anc, gmean, gmax, enc], axis=1)
        hb = jnp.maximum(_dot(comb, wb1T_ref[...]) + bb1_ref[...], 0.0)
        blog_ref[...] = _dot(hb, zb2_ref[...]) + bb2_ref[...]
        hs = jnp.maximum(_dot(comb, ws1T_ref[...]) + bs1_ref[...], 0.0)
        sev_ref[...] = _dot(hs, ws2T_ref[...]) + bs2_ref[...]


def _k3(ns, msgs, wsT, bs, aidx, feat, wfT, bf, wb1T, bb1, zb2, bb2p, ws1T,
        bs1, ws2T, bs2p):
    full = lambda *shape: pl.BlockSpec(shape, lambda i: tuple(0 for _ in shape))
    return pl.pallas_call(
        _k3_body,
        grid=(NB,),
        in_specs=[
            pl.BlockSpec((BN, H), lambda i: (i, 0)),
            pl.BlockSpec((NSC, BN, 32), lambda i: (0, i, 0)),
            full(H, H),
            full(1, H),
            pl.BlockSpec(memory_space=pltpu.SMEM),
            full(1, 48),
            full(48, 48),
            full(1, 48),
            full(240, 192),
            full(1, 192),
            full(192, 8),
            full(1, 8),
            full(240, H),
            full(1, H),
            full(H, 8),
            full(1, 8),
        ],
        out_specs=[
            pl.BlockSpec((1, 8), lambda i: (0, 0)),
            pl.BlockSpec((1, 8), lambda i: (0, 0)),
        ],
        out_shape=[
            jax.ShapeDtypeStruct((1, 8), f32),
            jax.ShapeDtypeStruct((1, 8), f32),
        ],
        scratch_shapes=[
            pltpu.VMEM((8, H), f32),
            pltpu.VMEM((8, H), f32),
            pltpu.VMEM((8, H), f32),
        ],
    )(ns, msgs, wsT, bs, aidx, feat, wfT, bf, wb1T, bb1, zb2, bb2p, ws1T, bs1,
      ws2T, bs2p)


def kernel(node_type_ids, edge_index, edge_type_ids, anchor_index, numeric,
           boolean, categorical, node_emb, Wself, bself, Wrel, ce0, ce1, ce2,
           Wf, bf, Wb1, bb1, Wb2, bb2, Ws1, bs1, Ws2, bs2):
    ids3 = node_type_ids.astype(i32).reshape(NB, 1, BN)
    src = edge_index[:, 0].astype(i32)
    dst = edge_index[:, 1].astype(i32)
    row = edge_type_ids.astype(i32) * N + src
    pad = EPAD - E
    ridx = jnp.concatenate([row, jnp.zeros((pad,), i32)]).reshape(
        EPAD // CH, CH)
    didx = jnp.concatenate([dst, jnp.full((pad,), DUMMY, i32)]).reshape(
        EPAD // CH, CH)
    zeros = jnp.zeros((ZR, 32), f32)

    wp = [jnp.transpose(Wrel[l], (2, 0, 1)).reshape(H, R * H) for l in
          range(L)]
    wsT = [Wself[l].T for l in range(L)]
    bs = [bself[l].reshape(1, H) for l in range(L)]

    aidx = jnp.asarray(anchor_index, i32).reshape(1, 1)
    feat = jnp.concatenate([
        numeric, boolean, ce0[categorical[0]], ce1[categorical[1]],
        ce2[categorical[2]]
    ]).reshape(1, 48).astype(f32)
    wfT = Wf.T
    bfr = bf.reshape(1, 48)
    wb1T = Wb1.reshape(3 * H, 240).T
    bb1r = bb1.reshape(1, 3 * H)
    kk = jnp.arange(3 * H)
    zb2 = jnp.zeros((3 * H, 8), f32).at[kk, kk // H].set(Wb2.reshape(3 * H))
    bb2p = jnp.zeros((1, 8), f32).at[0, :3].set(bb2)
    ws1T = Ws1.T
    bs1r = bs1.reshape(1, H)
    ws2T = jnp.zeros((H, 8), f32).at[:, :4].set(Ws2.T)
    bs2p = jnp.zeros((1, 8), f32).at[0, :4].set(bs2)

    ns, plo, phi = _k1(ids3, node_emb, wp[0])
    msgs = _sc_msgs(plo.reshape(R * N, 32), phi.reshape(R * N, 32), ridx,
                    didx, zeros)
    ns, plo, phi = _k2(ns, msgs, wsT[0], bs[0], wp[1])
    msgs = _sc_msgs(plo.reshape(R * N, 32), phi.reshape(R * N, 32), ridx,
                    didx, zeros)
    blog8, sev8 = _k3(ns, msgs, wsT[1], bs[1], aidx, feat, wfT, bfr, wb1T,
                      bb1r, zb2, bb2p, ws1T, bs1r, ws2T, bs2p)
    return jnp.concatenate([blog8[0, :3], sev8[0, :4]])


# speedup vs baseline: 13.7678x; 1.2605x over previous
import functools

import jax
import jax.numpy as jnp
from jax import lax
from jax.experimental import pallas as pl
from jax.experimental.pallas import tpu as pltpu
from jax.experimental.pallas import tpu_sc as plsc

N = 50000
E = 800000
H = 64
R = 4
L = 2

f32 = jnp.float32
i32 = jnp.int32

BN = 2000
NB = N // BN

NSC = 2
NTS = 16
CH = 64
KC = 28
PTC = 784
PT = PTC * CH
EPAD = NTS * PT
NGS = PTC // KC
NHB = 4
HSS = KC // NHB
NHS = PTC // NHB
A = 50176
ZR = A // NTS
DUMMY = N

_HI = lax.Precision.HIGHEST


def _dot(a, b):
    return jnp.dot(a, b, precision=_HI, preferred_element_type=f32)


def _k1_body(ids_ref, emb_ref, wp_ref, ns_ref, plo_ref, phi_ref):
    ids = ids_ref[0, 0, :]
    io = lax.broadcasted_iota(i32, (BN, H), 1)
    oh = jnp.where(ids[:, None] == io, 1.0, 0.0).astype(f32)
    ns = _dot(oh, emb_ref[...])
    ns_ref[...] = ns
    p = _dot(ns, wp_ref[...])
    for r in range(R):
        plo_ref[r] = p[:, r * H:r * H + 32]
        phi_ref[r] = p[:, r * H + 32:r * H + H]


def _k1(ids3, node_emb, wp0):
    return pl.pallas_call(
        _k1_body,
        grid=(NB,),
        in_specs=[
            pl.BlockSpec((1, 1, BN), lambda i: (i, 0, 0)),
            pl.BlockSpec((H, H), lambda i: (0, 0)),
            pl.BlockSpec((H, R * H), lambda i: (0, 0)),
        ],
        out_specs=[
            pl.BlockSpec((BN, H), lambda i: (i, 0)),
            pl.BlockSpec((R, BN, 32), lambda i: (0, i, 0)),
            pl.BlockSpec((R, BN, 32), lambda i: (0, i, 0)),
        ],
        out_shape=[
            jax.ShapeDtypeStruct((N, H), f32),
            jax.ShapeDtypeStruct((R, N, 32), f32),
            jax.ShapeDtypeStruct((R, N, 32), f32),
        ],
    )(ids3, node_emb, wp0)


def _sc_body(plo, phi, ridx, didx, zeros, out, acc, ridx_v, didx_v, rows_v,
             gsem, ssem):
    c = lax.axis_index("c")
    s = lax.axis_index("s")
    pltpu.sync_copy(zeros, acc.at[pl.ds(s * ZR, ZR)])
    plsc.subcore_barrier()

    def run(tbl):
        def stage(g, p):
            row0 = s * PTC + g * KC
            pltpu.sync_copy(ridx.at[pl.ds(row0, KC)], ridx_v.at[p])
            pltpu.sync_copy(didx.at[pl.ds(row0, KC)], didx_v.at[p])

        stage(0, 0)
        stage(1, 1)
        for k in range(NHB):
            pltpu.async_copy(tbl.at[ridx_v.at[0, k]],
                             rows_v.at[0, pl.ds(k * CH, CH)], gsem)

        def body(h, carry):
            b = h % 2
            p = (h // HSS) % 2
            ib = (h % HSS) * NHB
            for k in range(NHB):
                pltpu.make_async_copy(tbl.at[ridx_v.at[p, ib + k]],
                                      rows_v.at[b, pl.ds(k * CH, CH)],
                                      gsem).wait()
            for k in range(NHB):
                pltpu.async_copy(rows_v.at[b, pl.ds(k * CH, CH)],
                                 acc.at[didx_v.at[p, ib + k]], ssem, add=True)
            hm = h - 1
            bm = hm % 2
            pm = (hm // HSS) % 2
            ibm = (hm % HSS) * NHB

            @pl.when(h > 0)
            def _():
                for k in range(NHB):
                    pltpu.make_async_copy(rows_v.at[bm, pl.ds(k * CH, CH)],
                                          acc.at[didx_v.at[pm, ibm + k]],
                                          ssem).wait()

            g = h // HSS

            @pl.when((h == g * HSS) & (h > 0) & (g < NGS - 1))
            def _():
                stage(g + 1, (g + 1) % 2)

            hp = h + 1
            bp = hp % 2
            pp = (hp // HSS) % 2
            ibp = (hp % HSS) * NHB

            @pl.when(hp < NHS)
            def _():
                for k in range(NHB):
                    pltpu.async_copy(tbl.at[ridx_v.at[pp, ibp + k]],
                                     rows_v.at[bp, pl.ds(k * CH, CH)], gsem)

            return carry

        lax.fori_loop(0, NHS, body, 0)
        hl = NHS - 1
        bl = hl % 2
        pll = (hl // HSS) % 2
        ibl = (hl % HSS) * NHB
        for k in range(NHB):
            pltpu.make_async_copy(rows_v.at[bl, pl.ds(k * CH, CH)],
                                  acc.at[didx_v.at[pll, ibl + k]],
                                  ssem).wait()

    @pl.when(c == 0)
    def _():
        run(plo)

    @pl.when(c == 1)
    def _():
        run(phi)

    plsc.subcore_barrier()

    @pl.when(c == 0)
    def _():
        pltpu.sync_copy(acc.at[pl.ds(s * ZR, ZR)], out.at[0, pl.ds(s * ZR, ZR)])

    @pl.when(c == 1)
    def _():
        pltpu.sync_copy(acc.at[pl.ds(s * ZR, ZR)], out.at[1, pl.ds(s * ZR, ZR)])


_sc_msgs = pl.kernel(
    _sc_body,
    out_type=jax.ShapeDtypeStruct((NSC, A, 32), f32),
    mesh=plsc.VectorSubcoreMesh(
        core_axis_name="c", subcore_axis_name="s", num_cores=NSC,
        num_subcores=NTS),
    scratch_types=[
        pltpu.VMEM_SHARED((A, 32), f32),
        pltpu.VMEM((2, KC, CH), i32),
        pltpu.VMEM((2, KC, CH), i32),
        pltpu.VMEM((2, NHB * CH, 32), f32),
        pltpu.SemaphoreType.DMA,
        pltpu.SemaphoreType.DMA,
    ],
    compiler_params=pltpu.CompilerParams(use_tc_tiling_on_sc=False),
)


def _k2_body(ns_ref, m_ref, wsT_ref, bs_ref, wp_ref, nsn_ref, plo_ref,
             phi_ref):
    z = _dot(ns_ref[...], wsT_ref[...]) + bs_ref[...]
    msg = jnp.concatenate([m_ref[0], m_ref[1]], axis=1)
    nsn = jnp.maximum(z + msg, 0.0)
    nsn_ref[...] = nsn
    p = _dot(nsn, wp_ref[...])
    for r in range(R):
        plo_ref[r] = p[:, r * H:r * H + 32]
        phi_ref[r] = p[:, r * H + 32:r * H + H]


def _k2(ns, msgs, wsT, bs, wp):
    return pl.pallas_call(
        _k2_body,
        grid=(NB,),
        in_specs=[
            pl.BlockSpec((BN, H), lambda i: (i, 0)),
            pl.BlockSpec((NSC, BN, 32), lambda i: (0, i, 0)),
            pl.BlockSpec((H, H), lambda i: (0, 0)),
            pl.BlockSpec((1, H), lambda i: (0, 0)),
            pl.BlockSpec((H, R * H), lambda i: (0, 0)),
        ],
        out_specs=[
            pl.BlockSpec((BN, H), lambda i: (i, 0)),
            pl.BlockSpec((R, BN, 32), lambda i: (0, i, 0)),
            pl.BlockSpec((R, BN, 32), lambda i: (0, i, 0)),
        ],
        out_shape=[
            jax.ShapeDtypeStruct((N, H), f32),
            jax.ShapeDtypeStruct((R, N, 32), f32),
            jax.ShapeDtypeStruct((R, N, 32), f32),
        ],
    )(ns, msgs, wsT, bs, wp)


def _k3_body(ns_ref, m_ref, wsT_ref, bs_ref, aidx_ref, feat_ref, wfT_ref,
             bf_ref, wb1T_ref, bb1_ref, zb2_ref, bb2_ref, ws1T_ref, bs1_ref,
             ws2T_ref, bs2_ref, blog_ref, sev_ref, ssum, smax, sanc):
    i = pl.program_id(0)

    @pl.when(i == 0)
    def _():
        ssum[...] = jnp.zeros((8, H), f32)
        smax[...] = jnp.zeros((8, H), f32)
        sanc[...] = jnp.zeros((8, H), f32)
        blog_ref[...] = jnp.zeros((1, 8), f32)
        sev_ref[...] = jnp.zeros((1, 8), f32)

    z = _dot(ns_ref[...], wsT_ref[...]) + bs_ref[...]
    msg = jnp.concatenate([m_ref[0], m_ref[1]], axis=1)
    nsn = jnp.maximum(z + msg, 0.0)
    ssum[0:1, :] = ssum[0:1, :] + jnp.sum(nsn, axis=0, keepdims=True)
    smax[0:1, :] = jnp.maximum(smax[0:1, :], jnp.max(nsn, axis=0,
                                                     keepdims=True))
    aidx = aidx_ref[0, 0]
    rows = i * BN + lax.broadcasted_iota(i32, (BN, 1), 0)
    am = jnp.where(rows == aidx, 1.0, 0.0).astype(f32)
    sanc[0:1, :] = sanc[0:1, :] + jnp.sum(nsn * am, axis=0, keepdims=True)

    @pl.when(i == NB - 1)
    def _():
        gmean = ssum[0:1, :] * (1.0 / N)
        gmax = smax[0:1, :]
        anc = sanc[0:1, :]
        enc = jnp.maximum(_dot(feat_ref[...], wfT_ref[...]) + bf_ref[...],
                          0.0)
        comb = jnp.concatenate([anc, gmean, gmax, enc], axis=1)
        hb = jnp.maximum(_dot(comb, wb1T_ref[...]) + bb1_ref[...], 0.0)
        blog_ref[...] = _dot(hb, zb2_ref[...]) + bb2_ref[...]
        hs = jnp.maximum(_dot(comb, ws1T_ref[...]) + bs1_ref[...], 0.0)
        sev_ref[...] = _dot(hs, ws2T_ref[...]) + bs2_ref[...]


def _k3(ns, msgs, wsT, bs, aidx, feat, wfT, bf, wb1T, bb1, zb2, bb2p, ws1T,
        bs1, ws2T, bs2p):
    full = lambda *shape: pl.BlockSpec(shape, lambda i: tuple(0 for _ in shape))
    return pl.pallas_call(
        _k3_body,
        grid=(NB,),
        in_specs=[
            pl.BlockSpec((BN, H), lambda i: (i, 0)),
            pl.BlockSpec((NSC, BN, 32), lambda i: (0, i, 0)),
            full(H, H),
            full(1, H),
            pl.BlockSpec(memory_space=pltpu.SMEM),
            full(1, 48),
            full(48, 48),
            full(1, 48),
            full(240, 192),
            full(1, 192),
            full(192, 8),
            full(1, 8),
            full(240, H),
            full(1, H),
            full(H, 8),
            full(1, 8),
        ],
        out_specs=[
            pl.BlockSpec((1, 8), lambda i: (0, 0)),
            pl.BlockSpec((1, 8), lambda i: (0, 0)),
        ],
        out_shape=[
            jax.ShapeDtypeStruct((1, 8), f32),
            jax.ShapeDtypeStruct((1, 8), f32),
        ],
        scratch_shapes=[
            pltpu.VMEM((8, H), f32),
            pltpu.VMEM((8, H), f32),
            pltpu.VMEM((8, H), f32),
        ],
    )(ns, msgs, wsT, bs, aidx, feat, wfT, bf, wb1T, bb1, zb2, bb2p, ws1T, bs1,
      ws2T, bs2p)


def kernel(node_type_ids, edge_index, edge_type_ids, anchor_index, numeric,
           boolean, categorical, node_emb, Wself, bself, Wrel, ce0, ce1, ce2,
           Wf, bf, Wb1, bb1, Wb2, bb2, Ws1, bs1, Ws2, bs2):
    ids3 = node_type_ids.astype(i32).reshape(NB, 1, BN)
    src = edge_index[:, 0].astype(i32)
    dst = edge_index[:, 1].astype(i32)
    row = edge_type_ids.astype(i32) * N + src
    pad = EPAD - E
    ridx = jnp.concatenate([row, jnp.zeros((pad,), i32)]).reshape(
        EPAD // CH, CH)
    didx = jnp.concatenate([dst, jnp.full((pad,), DUMMY, i32)]).reshape(
        EPAD // CH, CH)
    zeros = jnp.zeros((ZR, 32), f32)

    wp = [jnp.transpose(Wrel[l], (2, 0, 1)).reshape(H, R * H) for l in
          range(L)]
    wsT = [Wself[l].T for l in range(L)]
    bs = [bself[l].reshape(1, H) for l in range(L)]

    aidx = jnp.asarray(anchor_index, i32).reshape(1, 1)
    feat = jnp.concatenate([
        numeric, boolean, ce0[categorical[0]], ce1[categorical[1]],
        ce2[categorical[2]]
    ]).reshape(1, 48).astype(f32)
    wfT = Wf.T
    bfr = bf.reshape(1, 48)
    wb1T = Wb1.reshape(3 * H, 240).T
    bb1r = bb1.reshape(1, 3 * H)
    kk = jnp.arange(3 * H)
    zb2 = jnp.zeros((3 * H, 8), f32).at[kk, kk // H].set(Wb2.reshape(3 * H))
    bb2p = jnp.zeros((1, 8), f32).at[0, :3].set(bb2)
    ws1T = Ws1.T
    bs1r = bs1.reshape(1, H)
    ws2T = jnp.zeros((H, 8), f32).at[:, :4].set(Ws2.T)
    bs2p = jnp.zeros((1, 8), f32).at[0, :4].set(bs2)

    ns, plo, phi = _k1(ids3, node_emb, wp[0])
    msgs = _sc_msgs(plo.reshape(R * N, 32), phi.reshape(R * N, 32), ridx,
                    didx, zeros)
    ns, plo, phi = _k2(ns, msgs, wsT[0], bs[0], wp[1])
    msgs = _sc_msgs(plo.reshape(R * N, 32), phi.reshape(R * N, 32), ridx,
                    didx, zeros)
    blog8, sev8 = _k3(ns, msgs, wsT[1], bs[1], aidx, feat, wfT, bfr, wb1T,
                      bb1r, zb2, bb2p, ws1T, bs1r, ws2T, bs2p)
    return jnp.concatenate([blog8[0, :3], sev8[0, :4]])
